# baseline (speedup 1.0000x reference)
import jax
import jax.numpy as jnp
from jax.experimental import pallas as pl
from jax.experimental.pallas import tpu as pltpu

_B = 32
_C_IN = 384
_HW = 1024
_HID = 512
_PC = 16
_NQ = 256
_NE = 8
_EPS = 1e-10


def _conv_logits_kernel(x_ref, w1_ref, b1_ref, w2_ref, b2_ref, w3_ref,
                        b3_ref, wg_ref, p_ref, out_ref):
    xb = x_ref[0]
    h1 = jnp.dot(w1_ref[...], xb.astype(jnp.bfloat16),
                 preferred_element_type=jnp.float32)
    h1 = jnp.maximum(h1 + b1_ref[...], 0.0)
    h2 = jnp.dot(w2_ref[...], h1.astype(jnp.bfloat16),
                 preferred_element_type=jnp.float32)
    h2 = jnp.maximum(h2 + b2_ref[...], 0.0)
    pool = jnp.dot(h2, p_ref[...], precision=jax.lax.Precision.HIGHEST)
    h3 = jnp.dot(w3_ref[...], pool,
                 precision=jax.lax.Precision.HIGHEST) + b3_ref[...]
    t = wg_ref[...] * h3[None, :, :]
    out_ref[...] = jnp.sum(t, axis=(1, 2))[None, None, :]


def _gating_kernel(l_ref, g_ref, i_ref, loss_ref):
    l = l_ref[...]
    eio = jax.lax.broadcasted_iota(jnp.int32, (_B, _NE), 1)
    m0 = jnp.max(l, axis=1, keepdims=True)
    i0 = jnp.min(jnp.where(l == m0, eio, _NE), axis=1, keepdims=True)
    lmask = jnp.where(eio == i0, -jnp.inf, l)
    m1 = jnp.max(lmask, axis=1, keepdims=True)
    i1 = jnp.min(jnp.where(lmask == m1, eio, _NE), axis=1, keepdims=True)
    e1 = jnp.exp(m1 - m0)
    s = 1.0 + e1
    g0 = 1.0 / s
    g1 = e1 / s
    g_ref[...] = jnp.concatenate([g0, g1], axis=1)
    i_ref[...] = jnp.concatenate([i0, i1], axis=1)
    oh0 = (eio == i0).astype(jnp.float32)
    oh1 = (eio == i1).astype(jnp.float32)
    imp = jnp.sum(oh0 * g0 + oh1 * g1, axis=0, keepdims=True)
    load = jnp.sum(oh0 * (g0 > 0.0).astype(jnp.float32)
                   + oh1 * (g1 > 0.0).astype(jnp.float32),
                   axis=0, keepdims=True)

    def cv_sq(v):
        m = jnp.mean(v)
        d = v - m
        var = jnp.sum(d * d) / (_NE - 1)
        return var / (m * m + _EPS)

    loss_ref[...] = (cv_sq(imp) + cv_sq(load))[None, None]


def kernel(x, W1, b1, W2, b2, W3, b3, w_gate):
    x3 = x.reshape(_B, _C_IN, _HW)
    w1b = W1.astype(jnp.bfloat16)
    w2b = W2.astype(jnp.bfloat16)
    b1c = b1.reshape(_HID, 1)
    b2c = b2.reshape(_PC, 1)
    b3c = b3.reshape(_PC, 1)
    w3r = W3.astype(jnp.bfloat16).astype(jnp.float32)
    wgr = (w_gate.astype(jnp.bfloat16).astype(jnp.float32)
           .reshape(_PC, _NQ, _NE).transpose(2, 0, 1))
    p_rows = jnp.arange(_HW, dtype=jnp.int32)[:, None]
    qmap = (p_rows // 32 // 2) * 16 + (p_rows % 32) // 2
    q_cols = jnp.arange(_NQ, dtype=jnp.int32)[None, :]
    pmat = jnp.where(qmap == q_cols, jnp.float32(0.25), jnp.float32(0.0))

    logits3 = pl.pallas_call(
        _conv_logits_kernel,
        grid=(_B,),
        in_specs=[
            pl.BlockSpec((1, _C_IN, _HW), lambda b: (b, 0, 0)),
            pl.BlockSpec((_HID, _C_IN), lambda b: (0, 0)),
            pl.BlockSpec((_HID, 1), lambda b: (0, 0)),
            pl.BlockSpec((_PC, _HID), lambda b: (0, 0)),
            pl.BlockSpec((_PC, 1), lambda b: (0, 0)),
            pl.BlockSpec((_PC, _PC), lambda b: (0, 0)),
            pl.BlockSpec((_PC, 1), lambda b: (0, 0)),
            pl.BlockSpec((_NE, _PC, _NQ), lambda b: (0, 0, 0)),
            pl.BlockSpec((_HW, _NQ), lambda b: (0, 0)),
        ],
        out_specs=pl.BlockSpec((1, 1, _NE), lambda b: (b, 0, 0)),
        out_shape=jax.ShapeDtypeStruct((_B, 1, _NE), jnp.float32),
    )(x3, w1b, b1c, w2b, b2c, w3r, b3c, wgr, pmat)

    logits = logits3.reshape(_B, _NE)

    gates, idx, loss = pl.pallas_call(
        _gating_kernel,
        out_shape=(
            jax.ShapeDtypeStruct((_B, 2), jnp.float32),
            jax.ShapeDtypeStruct((_B, 2), jnp.int32),
            jax.ShapeDtypeStruct((1, 1), jnp.float32),
        ),
    )(logits)

    return gates, idx, loss.reshape(())


# speedup vs baseline: 1.2436x; 1.2436x over previous
import jax
import jax.numpy as jnp
from jax.experimental import pallas as pl
from jax.experimental.pallas import tpu as pltpu

_B = 32
_C_IN = 384
_HW = 1024
_HID = 512
_PC = 16
_NQ = 256
_NE = 8
_EPS = 1e-10


def _conv_logits_kernel(x_ref, w1_ref, b1_ref, w2_ref, b2_ref, w3t_ref,
                        b3_ref, wg_ref, out_ref, g_scr, c_scr):
    b = pl.program_id(0)

    @pl.when(b == 0)
    def _prologue():
        p_idx = jax.lax.broadcasted_iota(jnp.int32, (_NQ, _HW), 1)
        q_idx = jax.lax.broadcasted_iota(jnp.int32, (_NQ, _HW), 0)
        qmap = (p_idx // 64) * 16 + (p_idx % 32) // 2
        e_mat = jnp.where(qmap == q_idx, jnp.float32(0.25), jnp.float32(0.0))
        hi = jax.lax.Precision.HIGHEST
        for e in range(_NE):
            wgt_e = wg_ref[e]
            m_e = jnp.dot(w3t_ref[...], wgt_e, precision=hi)
            g_scr[e] = jnp.dot(m_e, e_mat, precision=hi)
        s = jnp.sum(wg_ref[...], axis=2)
        c_scr[...] = jnp.dot(s, b3_ref[...], precision=hi)

    xb = x_ref[0]
    h1 = jnp.dot(w1_ref[...], xb, preferred_element_type=jnp.float32)
    h1 = jnp.maximum(h1 + b1_ref[...], 0.0)
    h2 = jnp.dot(w2_ref[...], h1.astype(jnp.bfloat16),
                 preferred_element_type=jnp.float32)
    h2 = jnp.maximum(h2 + b2_ref[...], 0.0)
    t = g_scr[...] * h2[None, :, :]
    lrow = jnp.sum(t, axis=(1, 2))[:, None] + c_scr[...]
    out_ref[...] = lrow.T[None, :, :]


def _gating_kernel(l_ref, g_ref, i_ref, loss_ref):
    l = l_ref[...]
    eio = jax.lax.broadcasted_iota(jnp.int32, (_B, _NE), 1)
    m0 = jnp.max(l, axis=1, keepdims=True)
    i0 = jnp.min(jnp.where(l == m0, eio, _NE), axis=1, keepdims=True)
    lmask = jnp.where(eio == i0, -jnp.inf, l)
    m1 = jnp.max(lmask, axis=1, keepdims=True)
    i1 = jnp.min(jnp.where(lmask == m1, eio, _NE), axis=1, keepdims=True)
    e1 = jnp.exp(m1 - m0)
    s = 1.0 + e1
    g0 = 1.0 / s
    g1 = e1 / s
    g_ref[...] = jnp.concatenate([g0, g1], axis=1)
    i_ref[...] = jnp.concatenate([i0, i1], axis=1)
    oh0 = (eio == i0).astype(jnp.float32)
    oh1 = (eio == i1).astype(jnp.float32)
    imp = jnp.sum(oh0 * g0 + oh1 * g1, axis=0, keepdims=True)
    load = jnp.sum(oh0 * (g0 > 0.0).astype(jnp.float32)
                   + oh1 * (g1 > 0.0).astype(jnp.float32),
                   axis=0, keepdims=True)

    def cv_sq(v):
        m = jnp.mean(v)
        d = v - m
        var = jnp.sum(d * d) / (_NE - 1)
        return var / (m * m + _EPS)

    loss_ref[...] = (cv_sq(imp) + cv_sq(load))[None, None]


def kernel(x, W1, b1, W2, b2, W3, b3, w_gate):
    x3 = x.reshape(_B, _C_IN, _HW).astype(jnp.bfloat16)
    w1b = W1.astype(jnp.bfloat16)
    w2b = W2.astype(jnp.bfloat16)
    b1c = b1.reshape(_HID, 1)
    b2c = b2.reshape(_PC, 1)
    b3c = b3.reshape(_PC, 1)
    w3t = W3.astype(jnp.bfloat16).astype(jnp.float32).T
    wgr = (w_gate.astype(jnp.bfloat16).astype(jnp.float32)
           .reshape(_PC, _NQ, _NE).transpose(2, 0, 1))

    logits3 = pl.pallas_call(
        _conv_logits_kernel,
        grid=(_B,),
        in_specs=[
            pl.BlockSpec((1, _C_IN, _HW), lambda b: (b, 0, 0)),
            pl.BlockSpec((_HID, _C_IN), lambda b: (0, 0)),
            pl.BlockSpec((_HID, 1), lambda b: (0, 0)),
            pl.BlockSpec((_PC, _HID), lambda b: (0, 0)),
            pl.BlockSpec((_PC, 1), lambda b: (0, 0)),
            pl.BlockSpec((_PC, _PC), lambda b: (0, 0)),
            pl.BlockSpec((_PC, 1), lambda b: (0, 0)),
            pl.BlockSpec((_NE, _PC, _NQ), lambda b: (0, 0, 0)),
        ],
        out_specs=pl.BlockSpec((1, 1, _NE), lambda b: (b, 0, 0)),
        out_shape=jax.ShapeDtypeStruct((_B, 1, _NE), jnp.float32),
        scratch_shapes=[
            pltpu.VMEM((_NE, _PC, _HW), jnp.float32),
            pltpu.VMEM((_NE, 1), jnp.float32),
        ],
    )(x3, w1b, b1c, w2b, b2c, w3t, b3c, wgr)

    logits = logits3.reshape(_B, _NE)

    gates, idx, loss = pl.pallas_call(
        _gating_kernel,
        out_shape=(
            jax.ShapeDtypeStruct((_B, 2), jnp.float32),
            jax.ShapeDtypeStruct((_B, 2), jnp.int32),
            jax.ShapeDtypeStruct((1, 1), jnp.float32),
        ),
    )(logits)

    return gates, idx, loss.reshape(())


# speedup vs baseline: 1.2444x; 1.0007x over previous
import jax
import jax.numpy as jnp
from jax.experimental import pallas as pl
from jax.experimental.pallas import tpu as pltpu

_B = 32
_C_IN = 384
_HW = 1024
_HID = 512
_PC = 16
_NQ = 256
_NE = 8
_EPS = 1e-10


def _fold_kernel(w3t_ref, b3_ref, wg_ref, g_ref, c_ref):
    p_idx = jax.lax.broadcasted_iota(jnp.int32, (_NQ, _HW), 1)
    q_idx = jax.lax.broadcasted_iota(jnp.int32, (_NQ, _HW), 0)
    qmap = (p_idx // 64) * 16 + (p_idx % 32) // 2
    e_mat = jnp.where(qmap == q_idx, jnp.float32(0.25), jnp.float32(0.0))
    hi = jax.lax.Precision.HIGHEST
    for e in range(_NE):
        m_e = jnp.dot(w3t_ref[...], wg_ref[e], precision=hi)
        g_ref[e] = jnp.dot(m_e, e_mat, precision=hi)
    s = jnp.sum(wg_ref[...], axis=2)
    c_ref[...] = jnp.dot(s, b3_ref[...], precision=hi)


def _conv_logits_kernel(x_ref, w1_ref, b1_ref, w2_ref, b2_ref, g_ref,
                        c_ref, out_ref):
    xb = x_ref[0].astype(jnp.bfloat16)
    h1 = jnp.dot(w1_ref[...], xb, preferred_element_type=jnp.float32)
    h1 = jnp.maximum(h1 + b1_ref[...], 0.0)
    h2 = jnp.dot(w2_ref[...], h1.astype(jnp.bfloat16),
                 preferred_element_type=jnp.float32)
    h2 = jnp.maximum(h2 + b2_ref[...], 0.0)
    t = g_ref[...] * h2[None, :, :]
    lrow = jnp.sum(t, axis=(1, 2))[:, None] + c_ref[...]
    out_ref[...] = lrow.T[None, :, :]


def _gating_kernel(l_ref, g_ref, i_ref, loss_ref):
    l = l_ref[...]
    eio = jax.lax.broadcasted_iota(jnp.int32, (_B, _NE), 1)
    m0 = jnp.max(l, axis=1, keepdims=True)
    i0 = jnp.min(jnp.where(l == m0, eio, _NE), axis=1, keepdims=True)
    lmask = jnp.where(eio == i0, -jnp.inf, l)
    m1 = jnp.max(lmask, axis=1, keepdims=True)
    i1 = jnp.min(jnp.where(lmask == m1, eio, _NE), axis=1, keepdims=True)
    e1 = jnp.exp(m1 - m0)
    s = 1.0 + e1
    g0 = 1.0 / s
    g1 = e1 / s
    g_ref[...] = jnp.concatenate([g0, g1], axis=1)
    i_ref[...] = jnp.concatenate([i0, i1], axis=1)
    oh0 = (eio == i0).astype(jnp.float32)
    oh1 = (eio == i1).astype(jnp.float32)
    imp = jnp.sum(oh0 * g0 + oh1 * g1, axis=0, keepdims=True)
    load = jnp.sum(oh0 * (g0 > 0.0).astype(jnp.float32)
                   + oh1 * (g1 > 0.0).astype(jnp.float32),
                   axis=0, keepdims=True)

    def cv_sq(v):
        m = jnp.mean(v)
        d = v - m
        var = jnp.sum(d * d) / (_NE - 1)
        return var / (m * m + _EPS)

    loss_ref[...] = (cv_sq(imp) + cv_sq(load))[None, None]


def kernel(x, W1, b1, W2, b2, W3, b3, w_gate):
    x3 = x.reshape(_B, _C_IN, _HW)
    w1b = W1.astype(jnp.bfloat16)
    w2b = W2.astype(jnp.bfloat16)
    b1c = b1.reshape(_HID, 1)
    b2c = b2.reshape(_PC, 1)
    b3c = b3.reshape(_PC, 1)
    w3t = W3.astype(jnp.bfloat16).astype(jnp.float32).T
    wgr = (w_gate.astype(jnp.bfloat16).astype(jnp.float32)
           .reshape(_PC, _NQ, _NE).transpose(2, 0, 1))

    g_full, consts = pl.pallas_call(
        _fold_kernel,
        out_shape=(
            jax.ShapeDtypeStruct((_NE, _PC, _HW), jnp.float32),
            jax.ShapeDtypeStruct((_NE, 1), jnp.float32),
        ),
    )(w3t, b3c, wgr)

    logits3 = pl.pallas_call(
        _conv_logits_kernel,
        grid=(_B,),
        in_specs=[
            pl.BlockSpec((1, _C_IN, _HW), lambda b: (b, 0, 0)),
            pl.BlockSpec((_HID, _C_IN), lambda b: (0, 0)),
            pl.BlockSpec((_HID, 1), lambda b: (0, 0)),
            pl.BlockSpec((_PC, _HID), lambda b: (0, 0)),
            pl.BlockSpec((_PC, 1), lambda b: (0, 0)),
            pl.BlockSpec((_NE, _PC, _HW), lambda b: (0, 0, 0)),
            pl.BlockSpec((_NE, 1), lambda b: (0, 0)),
        ],
        out_specs=pl.BlockSpec((1, 1, _NE), lambda b: (b, 0, 0)),
        out_shape=jax.ShapeDtypeStruct((_B, 1, _NE), jnp.float32),
        compiler_params=pltpu.CompilerParams(
            dimension_semantics=("parallel",),
        ),
    )(x3, w1b, b1c, w2b, b2c, g_full, consts)

    logits = logits3.reshape(_B, _NE)

    gates, idx, loss = pl.pallas_call(
        _gating_kernel,
        out_shape=(
            jax.ShapeDtypeStruct((_B, 2), jnp.float32),
            jax.ShapeDtypeStruct((_B, 2), jnp.int32),
            jax.ShapeDtypeStruct((1, 1), jnp.float32),
        ),
    )(logits)

    return gates, idx, loss.reshape(())
